# baseline (speedup 1.0000x reference)
import jax
import jax.numpy as jnp
from jax import lax
from jax.experimental import pallas as pl
from jax.experimental.pallas import tpu as pltpu

K = 1024
D = 64
B = 16
HW = 1024
T = 1024
BPS = 4
NTOK = B * HW
NSTEP = B // BPS
COMMIT = 0.25


def _vq_body(x_ref, emb_ref, enc_ref, q_ref, loss_ref, perp_ref,
             counts_ref, sse_ref):
    i = pl.program_id(0)

    @pl.when(i == 0)
    def _init():
        counts_ref[...] = jnp.zeros_like(counts_ref)
        sse_ref[0] = 0.0

    for s in range(BPS):
        _vq_block(x_ref.at[s], emb_ref, enc_ref.at[pl.ds(s * T, T), :],
                  q_ref.at[s], counts_ref, sse_ref)

    @pl.when(i == NSTEP - 1)
    def _fini():
        loss_ref[0, 0] = (1.0 + COMMIT) * sse_ref[0] / (NTOK * D)
        avg = counts_ref[...] * (1.0 / NTOK)
        perp_ref[0, 0] = jnp.exp(-jnp.sum(avg * jnp.log(avg + 1e-10)))


def _vq_block(x_ref, emb_ref, enc_ref, q_ref, counts_ref, sse_ref):
    x = x_ref[...]
    x2 = jnp.sum(x ** 2, axis=0)
    xd = x + x
    emb = emb_ref[...]
    e2 = jnp.sum(emb ** 2, axis=1)
    xe2 = lax.dot_general(xd, emb, (((0,), (1,)), ((), ())),
                          preferred_element_type=jnp.float32)
    scores = (x2[:, None] + e2[None, :]) - xe2
    minval = jnp.min(scores, axis=1)
    iota_k = lax.broadcasted_iota(jnp.int32, (T, K), 1)
    idx = jnp.min(jnp.where(scores == minval[:, None], iota_k, K), axis=1)
    enc = (iota_k == idx[:, None]).astype(jnp.float32)
    enc_ref[...] = enc
    counts_ref[...] += jnp.sum(enc, axis=0)
    qT = lax.dot_general(emb, enc, (((0,), (1,)), ((), ())),
                         preferred_element_type=jnp.float32)
    q_ref[...] = qT
    diff = qT - x
    sse_ref[0] += jnp.sum(diff * diff)


def kernel(inputs, embedding):
    xr = inputs.reshape(B, D, HW)
    enc, q, loss, perp = pl.pallas_call(
        _vq_body,
        grid=(NSTEP,),
        in_specs=[
            pl.BlockSpec((BPS, D, HW), lambda i: (i, 0, 0)),
            pl.BlockSpec((K, D), lambda i: (0, 0)),
        ],
        out_specs=[
            pl.BlockSpec((BPS * T, K), lambda i: (i, 0)),
            pl.BlockSpec((BPS, D, HW), lambda i: (i, 0, 0)),
            pl.BlockSpec((1, 1), lambda i: (0, 0), memory_space=pltpu.SMEM),
            pl.BlockSpec((1, 1), lambda i: (0, 0), memory_space=pltpu.SMEM),
        ],
        out_shape=[
            jax.ShapeDtypeStruct((NTOK, K), jnp.float32),
            jax.ShapeDtypeStruct((B, D, HW), jnp.float32),
            jax.ShapeDtypeStruct((1, 1), jnp.float32),
            jax.ShapeDtypeStruct((1, 1), jnp.float32),
        ],
        scratch_shapes=[
            pltpu.VMEM((K,), jnp.float32),
            pltpu.SMEM((1,), jnp.float32),
        ],
    )(xr, embedding)
    quantized = q.reshape(B, D, 32, 32)
    return (loss[0, 0], quantized, perp[0, 0], enc)


# speedup vs baseline: 1.0435x; 1.0435x over previous
import jax
import jax.numpy as jnp
from jax import lax
from jax.experimental import pallas as pl
from jax.experimental.pallas import tpu as pltpu

K = 1024
D = 64
B = 16
HW = 1024
T = 1024
BPS = 2
NTOK = B * HW
NSTEP = B // BPS
COMMIT = 0.25


def _vq_body(x_ref, emb_ref, enc_ref, q_ref, loss_ref, perp_ref,
             counts_ref, sse_ref):
    i = pl.program_id(0)

    @pl.when(i == 0)
    def _init():
        counts_ref[...] = jnp.zeros_like(counts_ref)
        sse_ref[0] = 0.0

    for s in range(BPS):
        _vq_block(x_ref.at[s], emb_ref, enc_ref.at[pl.ds(s * T, T), :],
                  q_ref.at[s], counts_ref, sse_ref)

    @pl.when(i == NSTEP - 1)
    def _fini():
        loss_ref[0, 0] = (1.0 + COMMIT) * sse_ref[0] / (NTOK * D)
        avg = counts_ref[...] * (1.0 / NTOK)
        perp_ref[0, 0] = jnp.exp(-jnp.sum(avg * jnp.log(avg + 1e-10)))


def _vq_block(x_ref, emb_ref, enc_ref, q_ref, counts_ref, sse_ref):
    x = x_ref[...]
    x2 = jnp.sum(x ** 2, axis=0)
    xd = x + x
    emb = emb_ref[...]
    e2 = jnp.sum(emb ** 2, axis=1)
    xe2 = lax.dot_general(xd, emb, (((0,), (1,)), ((), ())),
                          preferred_element_type=jnp.float32)
    scores = (x2[:, None] + e2[None, :]) - xe2
    minval = jnp.min(scores, axis=1)
    iota_k = lax.broadcasted_iota(jnp.int32, (T, K), 1)
    idx = jnp.min(jnp.where(scores == minval[:, None], iota_k, K), axis=1)
    enc = (iota_k == idx[:, None]).astype(jnp.float32)
    enc_ref[...] = enc
    counts_ref[...] += jnp.sum(enc, axis=0)
    qT = lax.dot_general(emb, enc, (((0,), (1,)), ((), ())),
                         preferred_element_type=jnp.float32)
    q_ref[...] = qT
    diff = qT - x
    sse_ref[0] += jnp.sum(diff * diff)


def kernel(inputs, embedding):
    xr = inputs.reshape(B, D, HW)
    enc, q, loss, perp = pl.pallas_call(
        _vq_body,
        grid=(NSTEP,),
        in_specs=[
            pl.BlockSpec((BPS, D, HW), lambda i: (i, 0, 0)),
            pl.BlockSpec((K, D), lambda i: (0, 0)),
        ],
        out_specs=[
            pl.BlockSpec((BPS * T, K), lambda i: (i, 0)),
            pl.BlockSpec((BPS, D, HW), lambda i: (i, 0, 0)),
            pl.BlockSpec((1, 1), lambda i: (0, 0), memory_space=pltpu.SMEM),
            pl.BlockSpec((1, 1), lambda i: (0, 0), memory_space=pltpu.SMEM),
        ],
        out_shape=[
            jax.ShapeDtypeStruct((NTOK, K), jnp.float32),
            jax.ShapeDtypeStruct((B, D, HW), jnp.float32),
            jax.ShapeDtypeStruct((1, 1), jnp.float32),
            jax.ShapeDtypeStruct((1, 1), jnp.float32),
        ],
        scratch_shapes=[
            pltpu.VMEM((K,), jnp.float32),
            pltpu.SMEM((1,), jnp.float32),
        ],
    )(xr, embedding)
    quantized = q.reshape(B, D, 32, 32)
    return (loss[0, 0], quantized, perp[0, 0], enc)
